# Initial kernel scaffold; baseline (speedup 1.0000x reference)
import jax
import jax.numpy as jnp
from jax.experimental import pallas as pl


def kernel(feat, edge_index, return_elist, W):
    raise NotImplementedError("write your pallas kernel here")


# speedup vs baseline: 4.6407x; 4.6407x over previous
import functools

import jax
import jax.numpy as jnp
from jax import lax
from jax.experimental import pallas as pl
from jax.experimental.pallas import tpu as pltpu
from jax.experimental.pallas import tpu_sc as plsc

NC = 2
NS = 16
NW = NC * NS
L = 16
CH = 128


def _cdiv(a, b):
    return (a + b - 1) // b


def _sc_mesh():
    return plsc.VectorSubcoreMesh(
        core_axis_name="c", subcore_axis_name="s",
        num_cores=NC, num_subcores=NS)


def _make_deg_kernel(n_pad, chunks):
    rows_per = n_pad // NS

    def body(idx_hbm, z_hbm, ones_hbm, out_hbm, idx_v, ones_v, deg_sh):
        c = lax.axis_index("c")
        s = lax.axis_index("s")
        base = s * rows_per
        pltpu.sync_copy(z_hbm, deg_sh.at[pl.ds(base, rows_per)])
        pltpu.sync_copy(ones_hbm, ones_v)
        pltpu.sync_copy(idx_hbm.at[c, s], idx_v)
        plsc.subcore_barrier()

        def step(j, carry):
            pltpu.sync_copy(ones_v, deg_sh.at[idx_v.at[j]], add=True)
            return carry

        lax.fori_loop(0, chunks, step, 0)
        plsc.subcore_barrier()
        pltpu.sync_copy(deg_sh.at[pl.ds(base, rows_per)],
                        out_hbm.at[c, pl.ds(base, rows_per)])

    return pl.kernel(
        body,
        out_type=jax.ShapeDtypeStruct((NC, n_pad, 128), jnp.float32),
        mesh=_sc_mesh(),
        scratch_types=[
            pltpu.VMEM((chunks, CH), jnp.int32),
            pltpu.VMEM((CH, 128), jnp.float32),
            pltpu.VMEM_SHARED((n_pad, 128), jnp.float32),
        ],
    )


def _make_agg_kernel(n_pad, d, chunks):
    rows_per = n_pad // NS

    def body(feat_hbm, src_hbm, dst_hbm, z_hbm, out_hbm,
             src_v, dst_v, row_v, agg_sh, sem):
        c = lax.axis_index("c")
        s = lax.axis_index("s")
        w = c * NS + s
        base = s * rows_per
        pltpu.sync_copy(z_hbm, agg_sh.at[pl.ds(base, rows_per)])
        pltpu.sync_copy(src_hbm.at[w], src_v)
        pltpu.sync_copy(dst_hbm.at[w], dst_v)
        plsc.subcore_barrier()

        def step(j, carry):
            pltpu.async_copy(feat_hbm.at[src_v.at[j]], row_v, sem).wait()
            pltpu.sync_copy(row_v, agg_sh.at[dst_v.at[j]], add=True)
            return carry

        lax.fori_loop(0, chunks, step, 0)
        plsc.subcore_barrier()
        pltpu.sync_copy(agg_sh.at[pl.ds(base, rows_per)],
                        out_hbm.at[c, pl.ds(base, rows_per)])

    return pl.kernel(
        body,
        out_type=jax.ShapeDtypeStruct((NC, n_pad, d), jnp.float32),
        mesh=_sc_mesh(),
        scratch_types=[
            pltpu.VMEM((chunks, CH), jnp.int32),
            pltpu.VMEM((chunks, CH), jnp.int32),
            pltpu.VMEM((CH, d), jnp.float32),
            pltpu.VMEM_SHARED((n_pad, d), jnp.float32),
            pltpu.SemaphoreType.DMA,
        ],
    )


def _scale_body(f_ref, d_ref, o_ref):
    deg = jnp.maximum(d_ref[...][:, 0:1], 1.0)
    o_ref[...] = f_ref[...] * lax.rsqrt(deg)


def _out_body(a_ref, d_ref, w_ref, o_ref):
    ssum = a_ref[0] + a_ref[1]
    res = jnp.dot(ssum, w_ref[...], preferred_element_type=jnp.float32)
    deg = jnp.maximum(d_ref[...][:, 0:1], 1.0)
    o_ref[...] = res * lax.rsqrt(deg)


def kernel(feat, edge_index, return_elist, W):
    n, d = feat.shape
    d_out = W.shape[1]
    e = edge_index.shape[1]
    n_pad = _cdiv(n + 1, NS * 8) * NS * 8
    rows_per = n_pad // NS
    chunks1 = _cdiv(e, NS * CH)
    chunks3 = _cdiv(e, NW * CH)
    e1 = chunks1 * NS * CH
    e3 = chunks3 * NW * CH

    src = edge_index[0]
    dst = edge_index[1]
    pad1 = jnp.full((e1 - e,), n, jnp.int32)
    pad3 = jnp.full((e3 - e,), n, jnp.int32)
    src1 = jnp.concatenate([src, pad1]).reshape(NS, chunks1, CH)
    dst1 = jnp.concatenate([dst, pad1]).reshape(NS, chunks1, CH)
    idx1 = jnp.stack([src1, dst1])
    src3 = jnp.concatenate([src, pad3]).reshape(NW, chunks3, CH)
    dst3 = jnp.concatenate([dst, pad3]).reshape(NW, chunks3, CH)

    zeros_l = jnp.zeros((rows_per, 128), jnp.float32)
    ones_l = jnp.ones((CH, 128), jnp.float32)
    zeros_d = jnp.zeros((rows_per, d), jnp.float32)

    degs = _make_deg_kernel(n_pad, chunks1)(idx1, zeros_l, ones_l)

    feat_pad = jnp.zeros((n_pad, d), feat.dtype).at[:n].set(feat)
    feat_src = pl.pallas_call(
        _scale_body,
        out_shape=jax.ShapeDtypeStruct((n_pad, d), jnp.float32),
    )(feat_pad, degs[0])

    agg2 = _make_agg_kernel(n_pad, d, chunks3)(feat_src, src3, dst3, zeros_d)

    rst = pl.pallas_call(
        _out_body,
        out_shape=jax.ShapeDtypeStruct((n_pad, d_out), jnp.float32),
    )(agg2, degs[1], W)
    return rst[:n]


# speedup vs baseline: 4.9874x; 1.0747x over previous
import functools

import jax
import jax.numpy as jnp
from jax import lax
from jax.experimental import pallas as pl
from jax.experimental.pallas import tpu as pltpu
from jax.experimental.pallas import tpu_sc as plsc

NC = 2
NS = 16
NW = NC * NS
L = 16
CH = 128


def _cdiv(a, b):
    return (a + b - 1) // b


def _sc_mesh():
    return plsc.VectorSubcoreMesh(
        core_axis_name="c", subcore_axis_name="s",
        num_cores=NC, num_subcores=NS)


def _make_deg_kernel(n_pad, chunks):
    rows_per = n_pad // NS

    def body(idx_hbm, z_hbm, ones_hbm, out_hbm, idx_v, ones_v, deg_sh):
        c = lax.axis_index("c")
        s = lax.axis_index("s")
        base = s * rows_per
        pltpu.sync_copy(z_hbm, deg_sh.at[pl.ds(base, rows_per)])
        pltpu.sync_copy(ones_hbm, ones_v)
        pltpu.sync_copy(idx_hbm.at[c, s], idx_v)
        plsc.subcore_barrier()

        def step(j, carry):
            pltpu.sync_copy(ones_v, deg_sh.at[idx_v.at[j]], add=True)
            return carry

        lax.fori_loop(0, chunks, step, 0)
        plsc.subcore_barrier()
        pltpu.sync_copy(deg_sh.at[pl.ds(base, rows_per)],
                        out_hbm.at[c, pl.ds(base, rows_per)])

    return pl.kernel(
        body,
        out_type=jax.ShapeDtypeStruct((NC, n_pad, 128), jnp.float32),
        mesh=_sc_mesh(),
        scratch_types=[
            pltpu.VMEM((chunks, CH), jnp.int32),
            pltpu.VMEM((CH, 128), jnp.float32),
            pltpu.VMEM_SHARED((n_pad, 128), jnp.float32),
        ],
    )


def _make_agg_kernel(n_pad, d, chunks):
    rows_per = n_pad // NS
    p0 = _cdiv(_cdiv(chunks, 2), 8) * 8
    passes = [(0, p0), (p0, chunks - p0)]
    win = max(m for _, m in passes)
    assert all(m >= 2 for _, m in passes)

    def body(feat_hbm, src_hbm, dst_hbm, z_hbm, out_hbm,
             src_v, dst_v, row0_v, row1_v, agg_sh, sem0, sem1):
        c = lax.axis_index("c")
        s = lax.axis_index("s")
        w = c * NS + s
        base = s * rows_per
        pltpu.sync_copy(z_hbm, agg_sh.at[pl.ds(base, rows_per)])
        plsc.subcore_barrier()

        for off, m in passes:
            pltpu.sync_copy(src_hbm.at[w, pl.ds(off, m)],
                            src_v.at[pl.ds(0, m)])
            pltpu.sync_copy(dst_hbm.at[w, pl.ds(off, m)],
                            dst_v.at[pl.ds(0, m)])
            pltpu.async_copy(feat_hbm.at[src_v.at[0]], row0_v, sem0)

            def step(jj, carry):
                j0 = 2 * jj
                pltpu.make_async_copy(feat_hbm.at[src_v.at[j0]], row0_v,
                                      sem0).wait()
                pltpu.async_copy(feat_hbm.at[src_v.at[j0 + 1]], row1_v, sem1)
                pltpu.sync_copy(row0_v, agg_sh.at[dst_v.at[j0]], add=True)
                pltpu.make_async_copy(feat_hbm.at[src_v.at[j0 + 1]], row1_v,
                                      sem1).wait()
                pltpu.async_copy(feat_hbm.at[src_v.at[j0 + 2]], row0_v, sem0)
                pltpu.sync_copy(row1_v, agg_sh.at[dst_v.at[j0 + 1]],
                                add=True)
                return carry

            lax.fori_loop(0, (m - 1) // 2, step, 0)
            if m % 2 == 1:
                pltpu.make_async_copy(feat_hbm.at[src_v.at[m - 1]], row0_v,
                                      sem0).wait()
                pltpu.sync_copy(row0_v, agg_sh.at[dst_v.at[m - 1]], add=True)
            else:
                pltpu.make_async_copy(feat_hbm.at[src_v.at[m - 2]], row0_v,
                                      sem0).wait()
                pltpu.async_copy(feat_hbm.at[src_v.at[m - 1]], row1_v, sem1)
                pltpu.sync_copy(row0_v, agg_sh.at[dst_v.at[m - 2]], add=True)
                pltpu.make_async_copy(feat_hbm.at[src_v.at[m - 1]], row1_v,
                                      sem1).wait()
                pltpu.sync_copy(row1_v, agg_sh.at[dst_v.at[m - 1]], add=True)

        plsc.subcore_barrier()
        pltpu.sync_copy(agg_sh.at[pl.ds(base, rows_per)],
                        out_hbm.at[c, pl.ds(base, rows_per)])

    return pl.kernel(
        body,
        out_type=jax.ShapeDtypeStruct((NC, n_pad, d), jnp.float32),
        mesh=_sc_mesh(),
        scratch_types=[
            pltpu.VMEM((win, CH), jnp.int32),
            pltpu.VMEM((win, CH), jnp.int32),
            pltpu.VMEM((CH, d), jnp.float32),
            pltpu.VMEM((CH, d), jnp.float32),
            pltpu.VMEM_SHARED((n_pad, d), jnp.float32),
            pltpu.SemaphoreType.DMA,
            pltpu.SemaphoreType.DMA,
        ],
    )


def _scale_body(f_ref, d_ref, o_ref):
    deg = jnp.maximum(d_ref[...][:, 0:1], 1.0)
    o_ref[...] = f_ref[...] * lax.rsqrt(deg)


def _out_body(a_ref, d_ref, w_ref, o_ref):
    ssum = a_ref[0] + a_ref[1]
    res = jnp.dot(ssum, w_ref[...], preferred_element_type=jnp.float32)
    deg = jnp.maximum(d_ref[...][:, 0:1], 1.0)
    o_ref[...] = res * lax.rsqrt(deg)


def kernel(feat, edge_index, return_elist, W):
    n, d = feat.shape
    d_out = W.shape[1]
    e = edge_index.shape[1]
    n_pad = _cdiv(n + 1, NS * 8) * NS * 8
    rows_per = n_pad // NS
    chunks1 = _cdiv(e, NS * CH)
    chunks3 = _cdiv(e, NW * CH)
    e1 = chunks1 * NS * CH
    e3 = chunks3 * NW * CH

    src = edge_index[0]
    dst = edge_index[1]
    pad1 = jnp.full((e1 - e,), n, jnp.int32)
    pad3 = jnp.full((e3 - e,), n, jnp.int32)
    src1 = jnp.concatenate([src, pad1]).reshape(NS, chunks1, CH)
    dst1 = jnp.concatenate([dst, pad1]).reshape(NS, chunks1, CH)
    idx1 = jnp.stack([src1, dst1])
    src3 = jnp.concatenate([src, pad3]).reshape(NW, chunks3, CH)
    dst3 = jnp.concatenate([dst, pad3]).reshape(NW, chunks3, CH)

    zeros_l = jnp.zeros((rows_per, 128), jnp.float32)
    ones_l = jnp.ones((CH, 128), jnp.float32)
    zeros_d = jnp.zeros((rows_per, d), jnp.float32)

    degs = _make_deg_kernel(n_pad, chunks1)(idx1, zeros_l, ones_l)

    feat_pad = jnp.zeros((n_pad, d), feat.dtype).at[:n].set(feat)
    feat_src = pl.pallas_call(
        _scale_body,
        out_shape=jax.ShapeDtypeStruct((n_pad, d), jnp.float32),
    )(feat_pad, degs[0])

    agg2 = _make_agg_kernel(n_pad, d, chunks3)(feat_src, src3, dst3, zeros_d)

    rst = pl.pallas_call(
        _out_body,
        out_shape=jax.ShapeDtypeStruct((n_pad, d_out), jnp.float32),
    )(agg2, degs[1], W)
    return rst[:n]
